# Initial kernel scaffold; baseline (speedup 1.0000x reference)
import jax
import jax.numpy as jnp
from jax.experimental import pallas as pl


def kernel(X, node_idx, edge_idx, num_e, W_wavelet):
    raise NotImplementedError("write your pallas kernel here")


# speedup vs baseline: 9.6835x; 9.6835x over previous
import functools

import jax
import jax.numpy as jnp
from jax import lax
from jax.experimental import pallas as pl
from jax.experimental.pallas import tpu as pltpu
from jax.experimental.pallas import tpu_sc as plsc

N = 10000
E = 20000
NNZ = 320000
F = 128
H = 64
N_P = 10240
E_P = 20480
NV_T = N_P // 16
NE_T = E_P // 16
NI_T = NNZ // 16
C = 80
G = NI_T // C
T16 = 16


def _sc_degrees(nidx_h, eidx_h, deg_v_out, deg_e_out,
                deg_v, deg_e, nidx, eidx, zvec, ones, obuf):
    c = lax.axis_index("c")
    s = lax.axis_index("s")

    pltpu.sync_copy(nidx_h.at[s], nidx)
    pltpu.sync_copy(eidx_h.at[s], eidx)

    z16 = jnp.zeros((16,), jnp.float32)
    one16 = jnp.zeros((16,), jnp.float32) + 1.0

    @pl.loop(0, NE_T // 16)
    def _zv(k):
        zvec[pl.ds(k * 16, 16)] = z16

    @pl.loop(0, C // 16)
    def _ones(k):
        ones[pl.ds(k * 16, 16)] = one16

    pltpu.sync_copy(zvec.at[pl.ds(0, NV_T)], deg_v.at[pl.ds(s * NV_T, NV_T)])
    pltpu.sync_copy(zvec, deg_e.at[pl.ds(s * NE_T, NE_T)])
    plsc.subcore_barrier()

    @pl.loop(0, G)
    def _deg(g):
        pltpu.sync_copy(ones, deg_v.at[nidx.at[g]], add=True)
        pltpu.sync_copy(ones, deg_e.at[eidx.at[g]], add=True)
    plsc.subcore_barrier()

    @pl.when(c == 0)
    def _():
        pltpu.sync_copy(deg_v.at[pl.ds(s * NV_T, NV_T)],
                        obuf.at[pl.ds(0, NV_T)])
        pltpu.sync_copy(obuf.at[pl.ds(0, NV_T)],
                        deg_v_out.at[pl.ds(s * NV_T, NV_T)])
        pltpu.sync_copy(deg_e.at[pl.ds(s * NE_T, NE_T)], obuf)
        pltpu.sync_copy(obuf, deg_e_out.at[pl.ds(s * NE_T, NE_T)])


_sc_degrees_call = functools.partial(
    pl.kernel,
    out_type=[
        jax.ShapeDtypeStruct((N_P,), jnp.float32),
        jax.ShapeDtypeStruct((E_P,), jnp.float32),
    ],
    mesh=plsc.VectorSubcoreMesh(core_axis_name="c", subcore_axis_name="s"),
    scratch_types=[
        pltpu.VMEM_SHARED((N_P,), jnp.float32),
        pltpu.VMEM_SHARED((E_P,), jnp.float32),
        pltpu.VMEM((G, C), jnp.int32),
        pltpu.VMEM((G, C), jnp.int32),
        pltpu.VMEM((NE_T,), jnp.float32),
        pltpu.VMEM((C,), jnp.float32),
        pltpu.VMEM((NE_T,), jnp.float32),
    ],
)(_sc_degrees)


BK = 50
NBK = G // BK
SR = 64


def _sc_diffusion(xn0, wfull_v, wfull_e, nidx_h, eidx_h,
                  levels, xn, eh,
                  acc,
                  nblk0, eblk0, nblk1, eblk1,
                  gbuf0, gbuf1, gbuf2, gbuf3, sbuf, wbuf, zbuf,
                  gsem0, gsem1, gsem2, gsem3, ssem0, ssem1, ssem2, ssem3,
                  bsem0, bsem1):
    c = lax.axis_index("c")
    s = lax.axis_index("s")

    z16 = jnp.zeros((16,), jnp.float32)

    @pl.loop(0, SR)
    def _zrow(r):
        for q in range(4):
            zbuf[r, pl.ds(q * 16, 16)] = z16

    @pl.loop(0, NV_T // SR)
    def _init(k):
        r0 = s * NV_T + k * SR
        pltpu.sync_copy(xn0.at[c, pl.ds(r0, SR), :], sbuf)
        pltpu.sync_copy(sbuf, xn.at[c, pl.ds(r0, SR), :])
    plsc.subcore_barrier()

    def _mul_rows():
        @pl.loop(0, SR)
        def _row(j):
            for q in range(4):
                sl = pl.ds(q * 16, 16)
                sbuf[j, sl] = sbuf[j, sl] * wbuf[j, sl]

    bufs = (gbuf0, gbuf1, gbuf2, gbuf3)
    gsems = (gsem0, gsem1, gsem2, gsem3)
    ssems = (ssem0, ssem1, ssem2, ssem3)

    def _drain(b, sem):
        pltpu.make_async_copy(xn0.at[0, pl.ds(0, C), :], bufs[b], sem).wait()

    def _zero_acc(nrows_t):
        @pl.loop(0, nrows_t // SR)
        def _z(k):
            pltpu.async_copy(
                zbuf, acc.at[pl.ds(s * nrows_t + k * SR, SR), :], gsem0)

        @pl.loop(0, nrows_t // SR)
        def _zw(k):
            pltpu.make_async_copy(
                xn0.at[0, pl.ds(0, SR), :], zbuf, gsem0).wait()

    def _pass(src_half0, src_half1, gather_is_n):
        sets = (((nblk0, eblk0) if gather_is_n else (eblk0, nblk0)),
                ((nblk1, eblk1) if gather_is_n else (eblk1, nblk1)))
        blks = ((nblk0, eblk0), (nblk1, eblk1))
        bsems = (bsem0, bsem1)

        def load_blk(bk, p):
            pltpu.async_copy(nidx_h.at[s, pl.ds(bk * BK, BK)], blks[p][0],
                             bsems[p])
            pltpu.async_copy(eidx_h.at[s, pl.ds(bk * BK, BK)], blks[p][1],
                             bsems[p])

        def wait_blk(p):
            for r in range(2):
                pltpu.make_async_copy(nidx_h.at[s, pl.ds(0, BK)],
                                      blks[p][r], bsems[p]).wait()

        def mk(p):
            gidx, sidx = sets[p]

            def ig(j, b):
                @pl.when(c == 0)
                def _():
                    pltpu.async_copy(src_half0.at[gidx.at[j]], bufs[b],
                                     gsems[b])

                @pl.when(c == 1)
                def _():
                    pltpu.async_copy(src_half1.at[gidx.at[j]], bufs[b],
                                     gsems[b])

            def sc(j, b):
                pltpu.async_copy(bufs[b], acc.at[sidx.at[j]], ssems[b],
                                 add=True)
            return ig, sc

        def run_blk(p, ig, sc):
            ig(0, 0)
            ig(1, 1)
            ig(2, 2)
            _drain(0, gsems[0])
            sc(0, 0)
            ig(3, 3)
            _drain(1, gsems[1])
            sc(1, 1)
            _drain(0, ssems[0])
            ig(4, 0)
            _drain(2, gsems[2])
            sc(2, 2)
            _drain(1, ssems[1])
            ig(5, 1)
            _drain(3, gsems[3])
            sc(3, 3)

            @pl.loop(0, (BK - 6) // 4)
            def _mid(i):
                for b4 in range(4):
                    j = 4 * i + 4 + b4
                    bn_ = (b4 + 2) % 4
                    _drain(bn_, ssems[bn_])
                    ig(j + 2, bn_)
                    _drain(b4, gsems[b4])
                    sc(j, b4)
            _drain(2, ssems[2])
            _drain(0, gsems[0])
            sc(BK - 2, 0)
            _drain(3, ssems[3])
            _drain(1, gsems[1])
            sc(BK - 1, 1)
            _drain(0, ssems[0])
            _drain(1, ssems[1])

        ig0, sc0 = mk(0)
        ig1, sc1 = mk(1)
        load_blk(0, 0)
        load_blk(1, 1)

        @pl.loop(0, 2)
        def _blkpair(i):
            wait_blk(0)
            run_blk(0, ig0, sc0)
            load_blk(2 * i + 2, 0)
            wait_blk(1)
            run_blk(1, ig1, sc1)

            @pl.when(i == 0)
            def _():
                load_blk(2 * i + 3, 1)
        wait_blk(0)
        run_blk(0, ig0, sc0)

    @pl.loop(0, T16)
    def _iter(it):
        _zero_acc(NE_T)
        plsc.subcore_barrier()

        _pass(xn.at[0], xn.at[1], True)
        plsc.subcore_barrier()

        @pl.loop(0, NE_T // SR)
        def _se(k):
            r0 = s * NE_T + k * SR
            pltpu.sync_copy(acc.at[pl.ds(r0, SR), :], sbuf)
            pltpu.sync_copy(wfull_e.at[pl.ds(r0, SR), :], wbuf)
            _mul_rows()
            pltpu.sync_copy(sbuf, eh.at[c, pl.ds(r0, SR), :])
        plsc.subcore_barrier()

        _zero_acc(NV_T)
        plsc.subcore_barrier()

        _pass(eh.at[0], eh.at[1], False)
        plsc.subcore_barrier()

        @pl.loop(0, NV_T // SR)
        def _ep(k):
            r0 = s * NV_T + k * SR
            pltpu.sync_copy(acc.at[pl.ds(r0, SR), :], sbuf)
            pltpu.sync_copy(sbuf, levels.at[it, c, pl.ds(r0, SR), :])
            pltpu.sync_copy(wfull_v.at[pl.ds(r0, SR), :], wbuf)
            _mul_rows()
            pltpu.sync_copy(sbuf, xn.at[c, pl.ds(r0, SR), :])
        plsc.subcore_barrier()


_sc_diffusion_call = functools.partial(
    pl.kernel,
    out_type=[
        jax.ShapeDtypeStruct((T16, 2, N_P, H), jnp.float32),
        jax.ShapeDtypeStruct((2, N_P, H), jnp.float32),
        jax.ShapeDtypeStruct((2, E_P, H), jnp.float32),
    ],
    mesh=plsc.VectorSubcoreMesh(core_axis_name="c", subcore_axis_name="s"),
    scratch_types=[
        pltpu.VMEM_SHARED((E_P, H), jnp.float32),
        pltpu.VMEM((BK, C), jnp.int32),
        pltpu.VMEM((BK, C), jnp.int32),
        pltpu.VMEM((BK, C), jnp.int32),
        pltpu.VMEM((BK, C), jnp.int32),
        pltpu.VMEM((C, H), jnp.float32),
        pltpu.VMEM((C, H), jnp.float32),
        pltpu.VMEM((C, H), jnp.float32),
        pltpu.VMEM((C, H), jnp.float32),
        pltpu.VMEM((SR, H), jnp.float32),
        pltpu.VMEM((SR, H), jnp.float32),
        pltpu.VMEM((SR, H), jnp.float32),
        pltpu.SemaphoreType.DMA,
        pltpu.SemaphoreType.DMA,
        pltpu.SemaphoreType.DMA,
        pltpu.SemaphoreType.DMA,
        pltpu.SemaphoreType.DMA,
        pltpu.SemaphoreType.DMA,
        pltpu.SemaphoreType.DMA,
        pltpu.SemaphoreType.DMA,
        pltpu.SemaphoreType.DMA,
        pltpu.SemaphoreType.DMA,
    ],
    compiler_params=pltpu.CompilerParams(use_tc_tiling_on_sc=False),
)(_sc_diffusion)


def _tc_expand_v(x2_ref, deg_ref, wfull_ref, xn0_ref):
    d = deg_ref[:, 0]
    w = jnp.where(d > 0.0, 1.0 / jnp.where(d > 0.0, d, 1.0), 0.0)
    wcol = w[:, None]
    wfull_ref[...] = jnp.broadcast_to(wcol, wfull_ref.shape)
    for h in range(2):
        xn0_ref[h] = x2_ref[h] * wcol


def _tc_expand_e(deg_ref, wfull_ref):
    d = deg_ref[:, 0]
    w = jnp.where(d > 0.0, 1.0 / jnp.where(d > 0.0, d, 1.0), 0.0)
    wfull_ref[...] = jnp.broadcast_to(w[:, None], wfull_ref.shape)


def _wavelet_tc(x2_ref, lev_ref, w_ref, out_ref):
    for j in range(6):
        for h in range(2):
            acc = x2_ref[h] * w_ref[j, 0]
            for l in range(1, 17):
                acc = acc + lev_ref[l - 1, h] * w_ref[j, l]
            out_ref[:, j, h * H:(h + 1) * H] = jnp.maximum(acc, 0.0)
            out_ref[:, j, F + h * H:F + (h + 1) * H] = jnp.maximum(-acc, 0.0)


def kernel(X, node_idx, edge_idx, num_e, W_wavelet):
    del num_e
    x2 = jnp.zeros((2, N_P, H), jnp.float32)
    x2 = x2.at[:, :N, :].set(jnp.transpose(X.reshape(N, 2, H), (1, 0, 2)))
    nidx_h = node_idx.reshape(16, G, C)
    eidx_h = edge_idx.reshape(16, G, C)

    deg_v, deg_e = _sc_degrees_call(nidx_h, eidx_h)

    bn = NV_T
    wfull_v, xn0 = pl.pallas_call(
        _tc_expand_v,
        grid=(N_P // bn,),
        in_specs=[
            pl.BlockSpec((2, bn, H), lambda i: (0, i, 0)),
            pl.BlockSpec((bn, 1), lambda i: (i, 0)),
        ],
        out_specs=[
            pl.BlockSpec((bn, H), lambda i: (i, 0)),
            pl.BlockSpec((2, bn, H), lambda i: (0, i, 0)),
        ],
        out_shape=[
            jax.ShapeDtypeStruct((N_P, H), jnp.float32),
            jax.ShapeDtypeStruct((2, N_P, H), jnp.float32),
        ],
    )(x2, deg_v[:, None])

    wfull_e = pl.pallas_call(
        _tc_expand_e,
        grid=(E_P // bn,),
        in_specs=[pl.BlockSpec((bn, 1), lambda i: (i, 0))],
        out_specs=pl.BlockSpec((bn, H), lambda i: (i, 0)),
        out_shape=jax.ShapeDtypeStruct((E_P, H), jnp.float32),
    )(deg_e[:, None])

    levels, _, _ = _sc_diffusion_call(xn0, wfull_v, wfull_e, nidx_h, eidx_h)

    s_pad = pl.pallas_call(
        _wavelet_tc,
        grid=(N_P // bn,),
        in_specs=[
            pl.BlockSpec((2, bn, H), lambda i: (0, i, 0)),
            pl.BlockSpec((T16, 2, bn, H), lambda i: (0, 0, i, 0)),
            pl.BlockSpec(memory_space=pltpu.MemorySpace.SMEM),
        ],
        out_specs=pl.BlockSpec((bn, 6, 2 * F), lambda i: (i, 0, 0)),
        out_shape=jax.ShapeDtypeStruct((N_P, 6, 2 * F), jnp.float32),
    )(x2, levels, W_wavelet)
    return s_pad[:N]
